# baseline (speedup 1.0000x reference)
import jax
import jax.numpy as jnp
from jax import lax
from jax.experimental import pallas as pl
from jax.experimental.pallas import tpu as pltpu
from jax.experimental.pallas import tpu_sc as plsc

_INFO = plsc.get_sparse_core_info()
_NC = _INFO.num_cores
_NS = _INFO.num_subcores
_NW = _NC * _NS
D = 64
_BPW = 512
_NSLOT = 8
_AHEAD = 6


def _read_key(sk_v, j, iota16):
    off = pl.multiple_of((j >> 4) * 16, 16)
    vec = sk_v[pl.ds(off, 16)]
    return jnp.sum(jnp.where(iota16 == (j & 15), vec, jnp.zeros_like(vec)))


def _stage_gather(skeys_hbm, pos_hbm, dlist_hbm, tbl, out_hbm, w,
                  sk_v, pos_v, dl_v, ring, rbs, semblk, semout, iota16):
    pltpu.sync_copy(skeys_hbm.at[w], sk_v)
    pltpu.sync_copy(pos_hbm.at[w], pos_v)
    pltpu.sync_copy(dlist_hbm.at[w], dl_v)

    def issue_step(nis, ncons):
        can = nis < ncons + _NSLOT - 1

        @pl.when(can)
        def _():
            bi = _read_key(dl_v, nis, iota16)
            off = pl.multiple_of(bi * 128, 128)
            pltpu.async_copy(tbl.at[:, pl.ds(off, 128)],
                             ring.at[nis & (_NSLOT - 1)], semblk)

        return nis + can.astype(jnp.int32)

    carries = (jnp.int32(0), jnp.int32(0))
    for t in range(4):
        rb = rbs[t % 2]
        if t >= 2:
            pltpu.make_async_copy(out_hbm.at[pl.ds(0, 128)], rb, semout).wait()

        def body(j2, c, t=t, rb=rb):
            nis, ncons = c
            nis = issue_step(nis, ncons)
            nis = issue_step(nis, ncons)
            v = _read_key(sk_v, t * 128 + j2, iota16)
            changed = v & 1

            @pl.when(changed != 0)
            def _():
                pltpu.make_async_copy(tbl.at[:, pl.ds(0, 128)],
                                      ring.at[0], semblk).wait()

            ncons = ncons + changed
            blk = ring.at[(ncons - 1) & (_NSLOT - 1)]
            colv = jnp.zeros_like(iota16) + ((v >> 1) & 127)
            rowv = jnp.zeros_like(iota16) + j2
            for q in range(4):
                dv = iota16 + q * 16
                x = plsc.load_gather(blk, [dv, colv])
                plsc.store_scatter(rb, [rowv, dv], x)
            return nis, ncons

        carries = lax.fori_loop(0, 128, body, carries)
        pltpu.async_copy(rb, out_hbm.at[pos_v.at[t]], semout)

    pltpu.make_async_copy(out_hbm.at[pl.ds(0, 128)], rbs[0], semout).wait()
    pltpu.make_async_copy(out_hbm.at[pl.ds(0, 128)], rbs[1], semout).wait()
    nis, ncons = carries

    def dbody(n):
        pltpu.make_async_copy(tbl.at[:, pl.ds(0, 128)],
                              ring.at[0], semblk).wait()
        return n + 1

    lax.while_loop(lambda n: n < nis, dbody, ncons)


def _sc_gather_body(su_hbm, pu_hbm, du_hbm, sm_hbm, pm_hbm, dm_hbm,
                    ut_hbm, mt_hbm,
                    uvw_out, mvw_out,
                    pos_v, ring, rb0, rb1, sk_v, dl_v, semblk, semout):
    wid = lax.axis_index("s") * _NC + lax.axis_index("c")
    iota16 = lax.iota(jnp.int32, 16)
    _stage_gather(su_hbm, pu_hbm, du_hbm, ut_hbm, uvw_out, wid,
                  sk_v, pos_v, dl_v, ring, (rb0, rb1), semblk, semout, iota16)
    _stage_gather(sm_hbm, pm_hbm, dm_hbm, mt_hbm, mvw_out, wid,
                  sk_v, pos_v, dl_v, ring, (rb0, rb1), semblk, semout, iota16)


def _sc_gather(su, pu, du, sm, pm, dm, utT, mtT):
    B = su.size
    mesh = plsc.VectorSubcoreMesh(core_axis_name="c", subcore_axis_name="s")
    fn = pl.kernel(
        _sc_gather_body,
        out_type=(
            jax.ShapeDtypeStruct((B, 128), jnp.float32),
            jax.ShapeDtypeStruct((B, 128), jnp.float32),
        ),
        mesh=mesh,
        scratch_types=[
            pltpu.VMEM((4, 128), jnp.int32),
            pltpu.VMEM((_NSLOT, D, 128), jnp.float32),
            pltpu.VMEM((128, 128), jnp.float32),
            pltpu.VMEM((128, 128), jnp.float32),
            pltpu.VMEM((_BPW,), jnp.int32),
            pltpu.VMEM((_BPW,), jnp.int32),
            pltpu.SemaphoreType.DMA,
            pltpu.SemaphoreType.DMA,
        ],
        compiler_params=pltpu.CompilerParams(use_tc_tiling_on_sc=True,
                                             needs_layout_passes=False),
    )
    return fn(su, pu, du, sm, pm, dm, utT, mtT)


def _prep(keys, iota):
    s, p = lax.sort_key_val(keys, iota)
    sw = s.reshape(_NW, _BPW)
    blocks = sw >> 7
    change = jnp.concatenate(
        [jnp.ones((_NW, 1), jnp.int32),
         (blocks[:, 1:] != blocks[:, :-1]).astype(jnp.int32)], axis=1)
    skc = (sw << 1) | change
    ordi = jnp.cumsum(change, axis=1) - 1
    dlist = jnp.zeros((_NW, _BPW), jnp.int32)
    dlist = dlist.at[jnp.arange(_NW)[:, None], ordi].set(blocks)
    return skc, p.reshape(_NW, 4, 128), dlist


def _tc_dense_body(uvw_ref, mvw_ref, g_ref, a_ref,
                   gt_ref, at_ref, gbt_ref, abt_ref,
                   w1_ref, b1_ref, w2_ref, b2_ref, w3_ref, b3_ref,
                   wo_ref, bo_ref, out_ref):
    uv = uvw_ref[...][:, :D]
    mv = mvw_ref[...][:, :D]
    g = g_ref[...]
    a = a_ref[...]
    gt = gt_ref[...]
    at = at_ref[...]
    gbt = gbt_ref[...]
    abt = abt_ref[...]
    gv = jnp.where(g == 0, gt[0:1, :], gt[1:2, :])
    gb = jnp.where(g == 0, gbt[0:1, :], gbt[1:2, :])
    na = at.shape[0]
    av = (a == 0).astype(jnp.float32) * at[0:1, :]
    ab = (a == 0).astype(jnp.float32) * abt[0:1, :]
    for i in range(1, na):
        sel = (a == i).astype(jnp.float32)
        av = av + sel * at[i:i + 1, :]
        ab = ab + sel * abt[i:i + 1, :]
    ga = gv + av
    dot = jnp.sum(uv * (mv + ga) + mv * ga + gv * av, axis=1, keepdims=True)
    x = dot + gb + ab
    h = jax.nn.relu(x * w1_ref[...] + b1_ref[...])
    h = jax.nn.relu(jnp.dot(h, w2_ref[...],
                            preferred_element_type=jnp.float32) + b2_ref[...])
    h = jax.nn.relu(jnp.dot(h, w3_ref[...],
                            preferred_element_type=jnp.float32) + b3_ref[...])
    o = jnp.dot(h, wo_ref[...], preferred_element_type=jnp.float32) + bo_ref[...]
    out_ref[...] = jax.nn.sigmoid(o)


def _tc_dense(uvw, mvw, g, a, gt, at, gbt, abt,
              W1, b1, W2, b2, W3, b3, Wo, bo, block):
    B = uvw.shape[0]
    grid = (B // block,)

    def row_spec(shape):
        return pl.BlockSpec((block,) + shape[1:],
                            lambda i: (i,) + (0,) * (len(shape) - 1))

    def full_spec(shape):
        return pl.BlockSpec(shape, lambda i: (0,) * len(shape))

    args = (uvw, mvw, g, a, gt, at, gbt, abt,
            W1, b1, W2, b2, W3, b3, Wo, bo)
    in_specs = [row_spec(uvw.shape), row_spec(mvw.shape),
                row_spec(g.shape), row_spec(a.shape)]
    in_specs += [full_spec(x.shape) for x in args[4:]]
    return pl.pallas_call(
        _tc_dense_body,
        grid=grid,
        in_specs=in_specs,
        out_specs=row_spec((B, 1)),
        out_shape=jax.ShapeDtypeStruct((B, 1), jnp.float32),
    )(*args)


def kernel(inputs, user_table, user_bias_table, movie_table, movie_bias_table,
           gender_table, gender_bias_table, age_table, age_bias_table,
           W1, b1, W2, b2, W3, b3, Wo, bo):
    B = inputs.shape[0]
    iota = jnp.arange(B, dtype=jnp.int32)
    su, pu, du = _prep(inputs[:, 0], iota)
    sm, pm, dm = _prep(inputs[:, 1], iota)
    g = inputs[:, 2:3]
    a = inputs[:, 3:4]
    uvw, mvw = _sc_gather(su, pu, du, sm, pm, dm,
                          user_table.T, movie_table.T)
    return _tc_dense(uvw, mvw, g, a,
                     gender_table, age_table,
                     gender_bias_table, age_bias_table,
                     W1, b1.reshape(1, -1), W2, b2.reshape(1, -1),
                     W3, b3.reshape(1, -1), Wo, bo.reshape(1, -1),
                     block=4096)


# speedup vs baseline: 1.6264x; 1.6264x over previous
import jax
import jax.numpy as jnp
from jax import lax
from jax.experimental import pallas as pl
from jax.experimental.pallas import tpu as pltpu
from jax.experimental.pallas import tpu_sc as plsc

_INFO = plsc.get_sparse_core_info()
_NC = _INFO.num_cores
_NS = _INFO.num_subcores
_NW = _NC * _NS
D = 64
_BPW = 512
_NSLOT = 8
_AHEAD = 6


def _read_key(sk_v, j, iota16):
    off = pl.multiple_of((j >> 4) * 16, 16)
    vec = sk_v[pl.ds(off, 16)]
    return jnp.sum(jnp.where(iota16 == (j & 15), vec, jnp.zeros_like(vec)))


def _stage_gather(skeys_hbm, pos_hbm, tbl, out_hbm, w,
                  sk_v, pos_v, ring, rbs, semblk, semout, iota16):
    pltpu.sync_copy(skeys_hbm.at[w], sk_v)
    pltpu.sync_copy(pos_hbm.at[w], pos_v)

    def issue_step(ji, prev_bi, nis, ncons):
        jr = jnp.minimum(ji, _BPW - 1)
        bi = _read_key(sk_v, jr, iota16) >> 7
        newb = jnp.logical_and(bi != prev_bi, ji < _BPW)
        cap = nis < ncons + _NSLOT - 1
        doit = jnp.logical_and(newb, cap)

        @pl.when(doit)
        def _():
            off = pl.multiple_of(bi * 128, 128)
            pltpu.async_copy(tbl.at[:, pl.ds(off, 128)],
                             ring.at[nis & (_NSLOT - 1)], semblk)

        nis = nis + doit.astype(jnp.int32)
        prev_bi = jnp.where(jnp.logical_or(doit, jnp.logical_not(newb)),
                            bi, prev_bi)
        ji = ji + jnp.where(jnp.logical_and(newb, jnp.logical_not(cap)),
                            0, 1)
        return ji, prev_bi, nis

    carries = (jnp.int32(0), jnp.int32(-1), jnp.int32(0),
               jnp.int32(-1), jnp.int32(0))
    for t in range(4):
        rb = rbs[t % 2]
        if t >= 2:
            pltpu.make_async_copy(out_hbm.at[pl.ds(0, 128)], rb, semout).wait()

        def body(j2, c, t=t, rb=rb):
            ji, prev_bi, nis, prev_bc, ncons = c
            ji, prev_bi, nis = issue_step(ji, prev_bi, nis, ncons)
            ji, prev_bi, nis = issue_step(ji, prev_bi, nis, ncons)
            s = _read_key(sk_v, t * 128 + j2, iota16)
            b = s >> 7
            changed = b != prev_bc

            @pl.when(changed)
            def _():
                pltpu.make_async_copy(tbl.at[:, pl.ds(0, 128)],
                                      ring.at[0], semblk).wait()

            ncons = ncons + changed.astype(jnp.int32)
            blk = ring.at[(ncons - 1) & (_NSLOT - 1)]
            colv = jnp.zeros_like(iota16) + (s & 127)
            rowv = jnp.zeros_like(iota16) + j2
            for q in range(4):
                dv = iota16 + q * 16
                v = plsc.load_gather(blk, [dv, colv])
                plsc.store_scatter(rb, [rowv, dv], v)
            return ji, prev_bi, nis, b, ncons

        carries = lax.fori_loop(0, 128, body, carries)
        pltpu.async_copy(rb, out_hbm.at[pos_v.at[t]], semout)

    pltpu.make_async_copy(out_hbm.at[pl.ds(0, 128)], rbs[0], semout).wait()
    pltpu.make_async_copy(out_hbm.at[pl.ds(0, 128)], rbs[1], semout).wait()
    nis, ncons = carries[2], carries[4]

    def dbody(n):
        pltpu.make_async_copy(tbl.at[:, pl.ds(0, 128)],
                              ring.at[0], semblk).wait()
        return n + 1

    lax.while_loop(lambda n: n < nis, dbody, ncons)


def _sc_gather_body(su_hbm, pu_hbm, sm_hbm, pm_hbm, ut_hbm, mt_hbm,
                    uvw_out, mvw_out,
                    pos_v, ring, rb0, rb1, sk_v, semblk, semout):
    wid = lax.axis_index("s") * _NC + lax.axis_index("c")
    iota16 = lax.iota(jnp.int32, 16)
    _stage_gather(su_hbm, pu_hbm, ut_hbm, uvw_out, wid,
                  sk_v, pos_v, ring, (rb0, rb1), semblk, semout, iota16)
    _stage_gather(sm_hbm, pm_hbm, mt_hbm, mvw_out, wid,
                  sk_v, pos_v, ring, (rb0, rb1), semblk, semout, iota16)


def _sc_gather(su, pu, sm, pm, utT, mtT):
    B = su.size
    mesh = plsc.VectorSubcoreMesh(core_axis_name="c", subcore_axis_name="s")
    fn = pl.kernel(
        _sc_gather_body,
        out_type=(
            jax.ShapeDtypeStruct((B, 128), jnp.float32),
            jax.ShapeDtypeStruct((B, 128), jnp.float32),
        ),
        mesh=mesh,
        scratch_types=[
            pltpu.VMEM((4, 128), jnp.int32),
            pltpu.VMEM((_NSLOT, D, 128), jnp.float32),
            pltpu.VMEM((128, 128), jnp.float32),
            pltpu.VMEM((128, 128), jnp.float32),
            pltpu.VMEM((_BPW,), jnp.int32),
            pltpu.SemaphoreType.DMA,
            pltpu.SemaphoreType.DMA,
        ],
        compiler_params=pltpu.CompilerParams(use_tc_tiling_on_sc=True,
                                             needs_layout_passes=False),
    )
    return fn(su, pu, sm, pm, utT, mtT)


def _tc_dense_body(uvw_ref, mvw_ref, g_ref, a_ref,
                   gt_ref, at_ref, gbt_ref, abt_ref,
                   w1_ref, b1_ref, w2_ref, b2_ref, w3_ref, b3_ref,
                   wo_ref, bo_ref, out_ref):
    uv = uvw_ref[...][:, :D]
    mv = mvw_ref[...][:, :D]
    g = g_ref[...]
    a = a_ref[...]
    gt = gt_ref[...]
    at = at_ref[...]
    gbt = gbt_ref[...]
    abt = abt_ref[...]
    gv = jnp.where(g == 0, gt[0:1, :], gt[1:2, :])
    gb = jnp.where(g == 0, gbt[0:1, :], gbt[1:2, :])
    na = at.shape[0]
    av = (a == 0).astype(jnp.float32) * at[0:1, :]
    ab = (a == 0).astype(jnp.float32) * abt[0:1, :]
    for i in range(1, na):
        sel = (a == i).astype(jnp.float32)
        av = av + sel * at[i:i + 1, :]
        ab = ab + sel * abt[i:i + 1, :]
    ga = gv + av
    dot = jnp.sum(uv * (mv + ga) + mv * ga + gv * av, axis=1, keepdims=True)
    x = dot + gb + ab
    h = jax.nn.relu(x * w1_ref[...] + b1_ref[...])
    h = jax.nn.relu(jnp.dot(h, w2_ref[...],
                            preferred_element_type=jnp.float32) + b2_ref[...])
    h = jax.nn.relu(jnp.dot(h, w3_ref[...],
                            preferred_element_type=jnp.float32) + b3_ref[...])
    o = jnp.dot(h, wo_ref[...], preferred_element_type=jnp.float32) + bo_ref[...]
    out_ref[...] = jax.nn.sigmoid(o)


def _tc_dense(uvw, mvw, g, a, gt, at, gbt, abt,
              W1, b1, W2, b2, W3, b3, Wo, bo, block):
    B = uvw.shape[0]
    grid = (B // block,)

    def row_spec(shape):
        return pl.BlockSpec((block,) + shape[1:],
                            lambda i: (i,) + (0,) * (len(shape) - 1))

    def full_spec(shape):
        return pl.BlockSpec(shape, lambda i: (0,) * len(shape))

    args = (uvw, mvw, g, a, gt, at, gbt, abt,
            W1, b1, W2, b2, W3, b3, Wo, bo)
    in_specs = [row_spec(uvw.shape), row_spec(mvw.shape),
                row_spec(g.shape), row_spec(a.shape)]
    in_specs += [full_spec(x.shape) for x in args[4:]]
    return pl.pallas_call(
        _tc_dense_body,
        grid=grid,
        in_specs=in_specs,
        out_specs=row_spec((B, 1)),
        out_shape=jax.ShapeDtypeStruct((B, 1), jnp.float32),
    )(*args)


def kernel(inputs, user_table, user_bias_table, movie_table, movie_bias_table,
           gender_table, gender_bias_table, age_table, age_bias_table,
           W1, b1, W2, b2, W3, b3, Wo, bo):
    B = inputs.shape[0]
    iota = jnp.arange(B, dtype=jnp.int32)
    su, pu = lax.sort_key_val(inputs[:, 0], iota)
    sm, pm = lax.sort_key_val(inputs[:, 1], iota)
    g = inputs[:, 2:3]
    a = inputs[:, 3:4]
    uvw, mvw = _sc_gather(su.reshape(_NW, _BPW), pu.reshape(_NW, 4, 128),
                          sm.reshape(_NW, _BPW), pm.reshape(_NW, 4, 128),
                          user_table.T, movie_table.T)
    return _tc_dense(uvw, mvw, g, a,
                     gender_table, age_table,
                     gender_bias_table, age_bias_table,
                     W1, b1.reshape(1, -1), W2, b2.reshape(1, -1),
                     W3, b3.reshape(1, -1), Wo, bo.reshape(1, -1),
                     block=4096)


# speedup vs baseline: 1.6341x; 1.0048x over previous
import jax
import jax.numpy as jnp
from jax import lax
from jax.experimental import pallas as pl
from jax.experimental.pallas import tpu as pltpu
from jax.experimental.pallas import tpu_sc as plsc

_INFO = plsc.get_sparse_core_info()
_NC = _INFO.num_cores
_NS = _INFO.num_subcores
_NW = _NC * _NS
D = 64
_BPW = 512
_NSLOT = 8
_AHEAD = 6


def _read_key(sk_v, j, iota16):
    off = pl.multiple_of((j >> 4) * 16, 16)
    vec = sk_v[pl.ds(off, 16)]
    return jnp.sum(jnp.where(iota16 == (j & 15), vec, jnp.zeros_like(vec)))


def _stage_gather(skeys_hbm, pos_hbm, tbl, out_hbm, w,
                  sk_v, pos_v, ring, rbs, semblk, semout, iota16):
    pltpu.sync_copy(skeys_hbm.at[w], sk_v)
    pltpu.sync_copy(pos_hbm.at[w], pos_v)

    def issue_step(ji, prev_bi, nis, ncons):
        jr = jnp.minimum(ji, _BPW - 1)
        bi = _read_key(sk_v, jr, iota16) >> 7
        newb = jnp.logical_and(bi != prev_bi, ji < _BPW)
        cap = nis < ncons + _NSLOT - 1
        doit = jnp.logical_and(newb, cap)

        @pl.when(doit)
        def _():
            off = pl.multiple_of(bi * 128, 128)
            pltpu.async_copy(tbl.at[:, pl.ds(off, 128)],
                             ring.at[nis & (_NSLOT - 1)], semblk)

        nis = nis + doit.astype(jnp.int32)
        prev_bi = jnp.where(jnp.logical_or(doit, jnp.logical_not(newb)),
                            bi, prev_bi)
        ji = ji + jnp.where(jnp.logical_and(newb, jnp.logical_not(cap)),
                            0, 1)
        return ji, prev_bi, nis

    carries = (jnp.int32(0), jnp.int32(-1), jnp.int32(0),
               jnp.int32(-1), jnp.int32(0))
    for t in range(4):
        rb = rbs[t % 2]
        if t >= 2:
            pltpu.make_async_copy(out_hbm.at[pl.ds(0, 128)], rb, semout).wait()

        def body(j2, c, t=t, rb=rb):
            ji, prev_bi, nis, prev_bc, ncons = c
            ji, prev_bi, nis = issue_step(ji, prev_bi, nis, ncons)
            ji, prev_bi, nis = issue_step(ji, prev_bi, nis, ncons)
            s = _read_key(sk_v, t * 128 + j2, iota16)
            b = s >> 7
            changed = b != prev_bc

            @pl.when(changed)
            def _():
                pltpu.make_async_copy(tbl.at[:, pl.ds(0, 128)],
                                      ring.at[0], semblk).wait()

            ncons = ncons + changed.astype(jnp.int32)
            blk = ring.at[(ncons - 1) & (_NSLOT - 1)]
            colv = jnp.zeros_like(iota16) + (s & 127)
            rowv = jnp.zeros_like(iota16) + j2
            for q in range(4):
                dv = iota16 + q * 16
                v = plsc.load_gather(blk, [dv, colv])
                plsc.store_scatter(rb, [rowv, dv], v)
            return ji, prev_bi, nis, b, ncons

        carries = lax.fori_loop(0, 128, body, carries)
        pltpu.async_copy(rb, out_hbm.at[pos_v.at[t]], semout)

    pltpu.make_async_copy(out_hbm.at[pl.ds(0, 128)], rbs[0], semout).wait()
    pltpu.make_async_copy(out_hbm.at[pl.ds(0, 128)], rbs[1], semout).wait()
    nis, ncons = carries[2], carries[4]

    def dbody(n):
        pltpu.make_async_copy(tbl.at[:, pl.ds(0, 128)],
                              ring.at[0], semblk).wait()
        return n + 1

    lax.while_loop(lambda n: n < nis, dbody, ncons)


def _sc_gather_body(su_hbm, pu_hbm, sm_hbm, pm_hbm, ut_hbm, mt_hbm,
                    uvw_out, mvw_out,
                    pos_v, ring, rb0, rb1, sk_v, semblk, semout):
    wid = lax.axis_index("s") * _NC + lax.axis_index("c")
    iota16 = lax.iota(jnp.int32, 16)
    _stage_gather(su_hbm, pu_hbm, ut_hbm, uvw_out, wid,
                  sk_v, pos_v, ring, (rb0, rb1), semblk, semout, iota16)
    _stage_gather(sm_hbm, pm_hbm, mt_hbm, mvw_out, wid,
                  sk_v, pos_v, ring, (rb0, rb1), semblk, semout, iota16)


def _sc_gather(su, pu, sm, pm, utT, mtT):
    B = su.size
    mesh = plsc.VectorSubcoreMesh(core_axis_name="c", subcore_axis_name="s")
    fn = pl.kernel(
        _sc_gather_body,
        out_type=(
            jax.ShapeDtypeStruct((B, 128), jnp.float32),
            jax.ShapeDtypeStruct((B, 128), jnp.float32),
        ),
        mesh=mesh,
        scratch_types=[
            pltpu.VMEM((4, 128), jnp.int32),
            pltpu.VMEM((_NSLOT, D, 128), jnp.float32),
            pltpu.VMEM((128, 128), jnp.float32),
            pltpu.VMEM((128, 128), jnp.float32),
            pltpu.VMEM((_BPW,), jnp.int32),
            pltpu.SemaphoreType.DMA,
            pltpu.SemaphoreType.DMA,
        ],
        compiler_params=pltpu.CompilerParams(use_tc_tiling_on_sc=True,
                                             needs_layout_passes=False),
    )
    return fn(su, pu, sm, pm, utT, mtT)


def _tc_dense_body(uvw_ref, mvw_ref, g_ref, a_ref,
                   gt_ref, at_ref, gbt_ref, abt_ref,
                   w1_ref, b1_ref, w2_ref, b2_ref, w3_ref, b3_ref,
                   wo_ref, bo_ref, out_ref):
    uv = uvw_ref[...][:, :D]
    mv = mvw_ref[...][:, :D]
    g = g_ref[...]
    a = a_ref[...]
    gt = gt_ref[...]
    at = at_ref[...]
    gbt = gbt_ref[...]
    abt = abt_ref[...]
    gv = jnp.where(g == 0, gt[0:1, :], gt[1:2, :])
    gb = jnp.where(g == 0, gbt[0:1, :], gbt[1:2, :])
    na = at.shape[0]
    av = (a == 0).astype(jnp.float32) * at[0:1, :]
    ab = (a == 0).astype(jnp.float32) * abt[0:1, :]
    for i in range(1, na):
        sel = (a == i).astype(jnp.float32)
        av = av + sel * at[i:i + 1, :]
        ab = ab + sel * abt[i:i + 1, :]
    ga = gv + av
    dot = jnp.sum(uv * (mv + ga) + mv * ga + gv * av, axis=1, keepdims=True)
    x = dot + gb + ab
    h = jax.nn.relu(x * w1_ref[...] + b1_ref[...])
    h = jax.nn.relu(jnp.dot(h, w2_ref[...],
                            preferred_element_type=jnp.float32) + b2_ref[...])
    h = jax.nn.relu(jnp.dot(h, w3_ref[...],
                            preferred_element_type=jnp.float32) + b3_ref[...])
    o = jnp.dot(h, wo_ref[...], preferred_element_type=jnp.float32) + bo_ref[...]
    out_ref[...] = jax.nn.sigmoid(o)


def _tc_dense(uvw, mvw, g, a, gt, at, gbt, abt,
              W1, b1, W2, b2, W3, b3, Wo, bo, block):
    B = uvw.shape[0]
    grid = (B // block,)

    def row_spec(shape):
        return pl.BlockSpec((block,) + shape[1:],
                            lambda i: (i,) + (0,) * (len(shape) - 1))

    def full_spec(shape):
        return pl.BlockSpec(shape, lambda i: (0,) * len(shape))

    args = (uvw, mvw, g, a, gt, at, gbt, abt,
            W1, b1, W2, b2, W3, b3, Wo, bo)
    in_specs = [row_spec(uvw.shape), row_spec(mvw.shape),
                row_spec(g.shape), row_spec(a.shape)]
    in_specs += [full_spec(x.shape) for x in args[4:]]
    return pl.pallas_call(
        _tc_dense_body,
        grid=grid,
        in_specs=in_specs,
        out_specs=row_spec((B, 1)),
        out_shape=jax.ShapeDtypeStruct((B, 1), jnp.float32),
    )(*args)


def kernel(inputs, user_table, user_bias_table, movie_table, movie_bias_table,
           gender_table, gender_bias_table, age_table, age_bias_table,
           W1, b1, W2, b2, W3, b3, Wo, bo):
    B = inputs.shape[0]
    iota = jnp.arange(B, dtype=jnp.int32)
    su, pu = lax.sort_key_val(inputs[:, 0], iota)
    sm, pm = lax.sort_key_val(inputs[:, 1], iota)
    g = inputs[:, 2:3]
    a = inputs[:, 3:4]
    uvw, mvw = _sc_gather(su.reshape(_NW, _BPW), pu.reshape(_NW, 4, 128),
                          sm.reshape(_NW, _BPW), pm.reshape(_NW, 4, 128),
                          user_table.T, movie_table.T)
    return _tc_dense(uvw, mvw, g, a,
                     gender_table, age_table,
                     gender_bias_table, age_bias_table,
                     W1, b1.reshape(1, -1), W2, b2.reshape(1, -1),
                     W3, b3.reshape(1, -1), Wo, bo.reshape(1, -1),
                     block=2048)
